# baseline (speedup 1.0000x reference)
import functools

import jax
import jax.numpy as jnp
from jax import lax
from jax.experimental import pallas as pl
from jax.experimental.pallas import tpu as pltpu
from jax.experimental.pallas import tpu_sc as plsc

N_NODES = 10000
N_EDGES = 320000
IN_CH = 128
OUT_CH = 128

NC, NS, LANES = 2, 16, 16
NT = NC * NS
N_PAD = 10240
HALF = N_PAD // 2
ROWS_PER_TILE = N_PAD // NS
ACC_ROWS_PER_TILE = HALF // NS
E_PAD = 327680
EPT_DEG = E_PAD // NT
GRPE = 1024
RCAP = 11264
CHUNK = 64
NSLOT = 2
ROW_BLK = 1024

_mesh = plsc.VectorSubcoreMesh(core_axis_name="c", subcore_axis_name="s")


@functools.partial(
    pl.kernel,
    out_type=jax.ShapeDtypeStruct((NC * N_PAD,), jnp.float32),
    mesh=_mesh,
    scratch_types=[
        pltpu.VMEM((EPT_DEG // 128, 128), jnp.int32),
        pltpu.VMEM((128,), jnp.float32),
        pltpu.VMEM_SHARED((N_PAD,), jnp.float32),
    ],
)
def _deg_kernel(dst2d_hbm, z1d_hbm, deg_out_hbm, dst_v, ones_v, deg_sh):
    c = lax.axis_index("c")
    s = lax.axis_index("s")
    t = c * NS + s
    pltpu.sync_copy(z1d_hbm, deg_sh.at[pl.ds(s * ROWS_PER_TILE, ROWS_PER_TILE)])
    pltpu.sync_copy(
        dst2d_hbm.at[pl.ds(t * (EPT_DEG // 128), EPT_DEG // 128)], dst_v
    )
    for i in range(128 // LANES):
        ones_v[pl.ds(i * LANES, LANES)] = jnp.full((LANES,), 1.0, jnp.float32)
    plsc.subcore_barrier()

    def body(j, carry):
        pltpu.sync_copy(ones_v, deg_sh.at[dst_v.at[j]], add=True)
        return carry

    lax.fori_loop(0, EPT_DEG // 128, body, 0)
    plsc.subcore_barrier()
    pltpu.sync_copy(
        deg_sh.at[pl.ds(s * ROWS_PER_TILE, ROWS_PER_TILE)],
        deg_out_hbm.at[pl.ds(c * N_PAD + s * ROWS_PER_TILE, ROWS_PER_TILE)],
    )


@functools.partial(
    pl.kernel,
    out_type=[
        jax.ShapeDtypeStruct((2 * NT * RCAP,), jnp.int32),
        jax.ShapeDtypeStruct((2 * NT * RCAP,), jnp.int32),
        jax.ShapeDtypeStruct((2 * NT * 2 * RCAP,), jnp.int32),
        jax.ShapeDtypeStruct((2 * NT * 16,), jnp.int32),
    ],
    mesh=_mesh,
    compiler_params=pltpu.CompilerParams(needs_layout_passes=False),
    scratch_types=[
        pltpu.VMEM((EPT_DEG,), jnp.int32),
        pltpu.VMEM((EPT_DEG,), jnp.int32),
        pltpu.VMEM((RCAP,), jnp.int32),
        pltpu.VMEM((RCAP,), jnp.int32),
        pltpu.VMEM((RCAP,), jnp.int32),
        pltpu.VMEM((RCAP,), jnp.int32),
        pltpu.VMEM((2 * RCAP,), jnp.int32),
        pltpu.VMEM((2 * RCAP,), jnp.int32),
        pltpu.VMEM((16,), jnp.int32),
    ],
)
def _part_kernel(src_hbm, dst_hbm, bsrc_out, bdst_out, bdx2_out, cnt_out,
                 sin_v, din_v, b0s_v, b0d_v, b1s_v, b1d_v, b0x_v, b1x_v,
                 cnt_v):
    c = lax.axis_index("c")
    s = lax.axis_index("s")
    t = c * NS + s
    pltpu.sync_copy(src_hbm.at[pl.ds(t * EPT_DEG, EPT_DEG)], sin_v)
    pltpu.sync_copy(dst_hbm.at[pl.ds(t * EPT_DEG, EPT_DEG)], din_v)

    def body(i, carry):
        o0, o1 = carry
        sv = sin_v[pl.ds(i * LANES, LANES)]
        dv = din_v[pl.ds(i * LANES, LANES)]
        n0 = jnp.max(plsc.all_reduce_population_count(dv < HALF))
        dvs, svs = plsc.sort_key_val(dv, sv)
        lane = lax.iota(jnp.int32, LANES)
        m0s = lane < n0
        idx0 = o0 + lane
        idx1 = o1 + lane - n0
        plsc.store_scatter(b0s_v, [idx0], svs, mask=m0s)
        plsc.store_scatter(b0d_v, [idx0], dvs, mask=m0s)
        d2lo = dvs * 2
        plsc.store_scatter(b0x_v, [idx0 * 2], d2lo, mask=m0s)
        plsc.store_scatter(b0x_v, [idx0 * 2 + 1], d2lo + 1, mask=m0s)
        m1s = jnp.logical_not(m0s)
        dloc = dvs - HALF
        plsc.store_scatter(b1s_v, [idx1], svs, mask=m1s)
        plsc.store_scatter(b1d_v, [idx1], dloc, mask=m1s)
        d2lo1 = dloc * 2
        plsc.store_scatter(b1x_v, [idx1 * 2], d2lo1, mask=m1s)
        plsc.store_scatter(b1x_v, [idx1 * 2 + 1], d2lo1 + 1, mask=m1s)
        return (o0 + n0, o1 + (LANES - n0))

    o0, o1 = lax.fori_loop(0, EPT_DEG // LANES, body,
                           (jnp.int32(0), jnp.int32(0)))

    jsrc = jnp.full((LANES,), N_NODES, jnp.int32)
    jdst = jnp.zeros((LANES,), jnp.int32)

    lane = lax.iota(jnp.int32, LANES)
    jx2 = lane % 2

    def padb(i, carry):
        po0, po1 = carry
        b0s_v[pl.ds(po0 + i * LANES, LANES)] = jsrc
        b0d_v[pl.ds(po0 + i * LANES, LANES)] = jdst
        b1s_v[pl.ds(po1 + i * LANES, LANES)] = jsrc
        b1d_v[pl.ds(po1 + i * LANES, LANES)] = jdst
        b0x_v[pl.ds(2 * po0 + 2 * i * LANES, LANES)] = jx2
        b0x_v[pl.ds(2 * po0 + 2 * i * LANES + LANES, LANES)] = jx2
        b1x_v[pl.ds(2 * po1 + 2 * i * LANES, LANES)] = jx2
        b1x_v[pl.ds(2 * po1 + 2 * i * LANES + LANES, LANES)] = jx2
        return carry

    lax.fori_loop(0, GRPE // LANES, padb, (o0, o1))
    p0 = ((o0 + GRPE - 1) // GRPE) * GRPE
    p1 = ((o1 + GRPE - 1) // GRPE) * GRPE

    cnt_v[...] = jnp.full((16,), 1, jnp.int32) * p0
    pltpu.sync_copy(cnt_v, cnt_out.at[pl.ds((0 * NT + t) * 16, 16)])
    cnt_v[...] = jnp.full((16,), 1, jnp.int32) * p1
    pltpu.sync_copy(cnt_v, cnt_out.at[pl.ds((1 * NT + t) * 16, 16)])
    pltpu.sync_copy(b0s_v, bsrc_out.at[pl.ds((0 * NT + t) * RCAP, RCAP)])
    pltpu.sync_copy(b0d_v, bdst_out.at[pl.ds((0 * NT + t) * RCAP, RCAP)])
    pltpu.sync_copy(b1s_v, bsrc_out.at[pl.ds((1 * NT + t) * RCAP, RCAP)])
    pltpu.sync_copy(b1d_v, bdst_out.at[pl.ds((1 * NT + t) * RCAP, RCAP)])
    pltpu.sync_copy(
        b0x_v, bdx2_out.at[pl.ds((0 * NT + t) * 2 * RCAP, 2 * RCAP)])
    pltpu.sync_copy(
        b1x_v, bdx2_out.at[pl.ds((1 * NT + t) * 2 * RCAP, 2 * RCAP)])


def _make_seg_sum(interleave):
    ipd = 2 if interleave else 1
    g_chunks = GRPE // CHUNK
    rounds = g_chunks // NSLOT

    @functools.partial(
        pl.kernel,
        out_type=jax.ShapeDtypeStruct((NC * ipd * HALF, 128), jnp.float32),
        mesh=_mesh,
        scratch_types=[
            pltpu.VMEM((GRPE,), jnp.int32),
            pltpu.VMEM((GRPE // CHUNK, ipd * CHUNK), jnp.int32),
            [pltpu.VMEM((ipd * CHUNK, 128), jnp.float32) for _ in range(NSLOT)],
            pltpu.VMEM((16,), jnp.int32),
            pltpu.VMEM_SHARED((ipd * HALF, 128), jnp.float32),
            [pltpu.SemaphoreType.DMA for _ in range(NSLOT)],
            [pltpu.SemaphoreType.DMA for _ in range(NSLOT)],
        ],
    )
    def seg(hs_hbm, bsrc_hbm, bdst2d_hbm, cnt_hbm, zw_hbm, acc_out_hbm,
            src_v, dst_v, rows, cnt_v, acc_sh, gsem, ssem):
        c = lax.axis_index("c")
        s = lax.axis_index("s")
        arpt = ipd * ACC_ROWS_PER_TILE
        pltpu.sync_copy(zw_hbm, acc_sh.at[pl.ds(s * arpt, arpt)])
        plsc.subcore_barrier()

        def gather(j, b):
            dst = rows[b].reshape(CHUNK, 2, 128) if interleave else rows[b]
            return pltpu.make_async_copy(
                hs_hbm.at[src_v.at[pl.ds(j * CHUNK, CHUNK)]], dst, gsem[b]
            )

        def scatter(j, b):
            return pltpu.make_async_copy(
                rows[b], acc_sh.at[dst_v.at[j]], ssem[b]
            )

        for r_i in range(2):
            r = 2 * s + r_i
            base_e = (c * NT + r) * RCAP
            base_row = (c * NT + r) * (RCAP // CHUNK)
            pltpu.sync_copy(cnt_hbm.at[pl.ds((c * NT + r) * 16, 16)], cnt_v)
            n_grp = cnt_v[...][0] // GRPE

            def group(g, carry):
                @pl.when(g < n_grp)
                def _():
                    pltpu.sync_copy(
                        bsrc_hbm.at[pl.ds(base_e + g * GRPE, GRPE)], src_v)
                    pltpu.sync_copy(
                        bdst2d_hbm.at[pl.ds(base_row + g * (GRPE // CHUNK),
                                            GRPE // CHUNK)], dst_v)
                    for b in range(NSLOT):
                        gather(b, b).start()

                    def body(k, carry2):
                        j0 = k * NSLOT
                        for b in range(NSLOT):
                            gather(j0 + b, b).wait()
                            scatter(j0 + b, b).start(add=True)

                        @pl.when(k < rounds - 1)
                        def _():
                            for b in range(NSLOT):
                                scatter(j0 + b, b).wait()
                                gather(j0 + NSLOT + b, b).start()
                        return carry2

                    lax.fori_loop(0, rounds, body, 0)
                    for b in range(NSLOT):
                        scatter(g_chunks - NSLOT + b, b).wait()
                return carry

            lax.fori_loop(0, RCAP // GRPE, group, 0)

        plsc.subcore_barrier()
        pltpu.sync_copy(
            acc_sh.at[pl.ds(s * arpt, arpt)],
            acc_out_hbm.at[pl.ds(c * ipd * HALF + s * arpt, arpt)],
        )

    return seg


_seg_sum_l1 = _make_seg_sum(True)
_seg_sum_l2 = _make_seg_sum(False)


def _mm1_body(x_ref, w1_ref, deg_ref, hs_ref, dis_ref):
    deg = deg_ref[0] + deg_ref[1] + 1.0
    dis = lax.rsqrt(deg)
    dis_ref[...] = dis
    h = jnp.dot(x_ref[...], w1_ref[...], preferred_element_type=jnp.float32)
    hs_ref[...] = h * dis[:, None]


def _mm2_body(acc_ref, hs_ref, dis_ref, b1_ref, w2_ref, hs2_ref):
    dis = dis_ref[...]
    h = jax.nn.relu(dis[:, None] * (acc_ref[...] + hs_ref[...])
                    + b1_ref[...][None, :])
    hs2 = jnp.dot(h, w2_ref[...], preferred_element_type=jnp.float32)
    hs2 = hs2 * dis[:, None]
    row = pl.program_id(0) * ROW_BLK + lax.broadcasted_iota(
        jnp.int32, (ROW_BLK, 1), 0)
    hs2_ref[...] = jnp.where(row < N_NODES, hs2, 0.0)


def _fin_body(acc_ref, hs_ref, dis_ref, b2_ref, out_ref):
    dis = dis_ref[...]
    out_ref[...] = (dis[:, None] * (acc_ref[...] + hs_ref[...])
                    + b2_ref[...][None, :])


def _row_grid():
    return N_PAD // ROW_BLK


def _tc_mm1(x_pad, W1, deg2):
    return pl.pallas_call(
        _mm1_body,
        grid=(_row_grid(),),
        in_specs=[
            pl.BlockSpec((ROW_BLK, IN_CH), lambda i: (i, 0)),
            pl.BlockSpec((IN_CH, 256), lambda i: (0, 0)),
            pl.BlockSpec((2, ROW_BLK), lambda i: (0, i)),
        ],
        out_specs=[
            pl.BlockSpec((ROW_BLK, 256), lambda i: (i, 0)),
            pl.BlockSpec((ROW_BLK,), lambda i: (i,)),
        ],
        out_shape=[
            jax.ShapeDtypeStruct((N_PAD, 256), jnp.float32),
            jax.ShapeDtypeStruct((N_PAD,), jnp.float32),
        ],
    )(x_pad, W1, deg2)


def _tc_mm2(acc1, hs1, dis, b1, W2):
    return pl.pallas_call(
        _mm2_body,
        grid=(_row_grid(),),
        in_specs=[
            pl.BlockSpec((ROW_BLK, 256), lambda i: (i, 0)),
            pl.BlockSpec((ROW_BLK, 256), lambda i: (i, 0)),
            pl.BlockSpec((ROW_BLK,), lambda i: (i,)),
            pl.BlockSpec((256,), lambda i: (0,)),
            pl.BlockSpec((256, 128), lambda i: (0, 0)),
        ],
        out_specs=pl.BlockSpec((ROW_BLK, 128), lambda i: (i, 0)),
        out_shape=jax.ShapeDtypeStruct((N_PAD, 128), jnp.float32),
    )(acc1, hs1, dis, b1, W2)


def _tc_fin(acc2, hs2, dis, b2):
    return pl.pallas_call(
        _fin_body,
        grid=(_row_grid(),),
        in_specs=[
            pl.BlockSpec((ROW_BLK, 128), lambda i: (i, 0)),
            pl.BlockSpec((ROW_BLK, 128), lambda i: (i, 0)),
            pl.BlockSpec((ROW_BLK,), lambda i: (i,)),
            pl.BlockSpec((OUT_CH,), lambda i: (0,)),
        ],
        out_specs=pl.BlockSpec((ROW_BLK, OUT_CH), lambda i: (i, 0)),
        out_shape=jax.ShapeDtypeStruct((N_PAD, OUT_CH), jnp.float32),
    )(acc2, hs2, dis, b2)


def kernel(x, edge_index, W1, b1, W2, b2):
    ei = edge_index.astype(jnp.int32)
    pad_e = E_PAD - N_EDGES
    src = jnp.concatenate([ei[0], jnp.full((pad_e,), N_NODES, jnp.int32)])
    dst = jnp.concatenate([ei[1], jnp.full((pad_e,), N_NODES, jnp.int32)])
    dst2d = dst.reshape(E_PAD // 128, 128)

    x_pad = jnp.pad(x, ((0, N_PAD - N_NODES), (0, 0)))
    z1d = jnp.zeros((ROWS_PER_TILE,), jnp.float32)
    zl1 = jnp.zeros((2 * ACC_ROWS_PER_TILE, 128), jnp.float32)
    zl2 = jnp.zeros((ACC_ROWS_PER_TILE, 128), jnp.float32)

    bsrc, bdst, bdx2, cnts = _part_kernel(src, dst)
    bdst2d = bdst.reshape(2 * NT * RCAP // CHUNK, CHUNK)
    bdx2_2d = bdx2.reshape(2 * NT * 2 * RCAP // (2 * CHUNK), 2 * CHUNK)
    deg2 = _deg_kernel(dst2d, z1d).reshape(2, N_PAD)

    hs1, dis = _tc_mm1(x_pad, W1, deg2)
    hs1_3d = hs1.reshape(N_PAD, 2, 128)
    acc1 = _seg_sum_l1(hs1_3d, bsrc, bdx2_2d, cnts, zl1).reshape(N_PAD, 256)

    hs2 = _tc_mm2(acc1, hs1, dis, b1, W2)
    acc2 = _seg_sum_l2(hs2, bsrc, bdst2d, cnts, zl2)

    out = _tc_fin(acc2, hs2, dis, b2)
    return out[:N_NODES]


# speedup vs baseline: 5.3193x; 5.3193x over previous
import functools

import jax
import jax.numpy as jnp
from jax import lax
from jax.experimental import pallas as pl
from jax.experimental.pallas import tpu as pltpu
from jax.experimental.pallas import tpu_sc as plsc

N_NODES = 10000
N_EDGES = 320000
IN_CH = 128
OUT_CH = 128

NC, NS, LANES = 2, 16, 16
NT = NC * NS
N_PAD = 10240
HALF = N_PAD // 2
ROWS_PER_TILE = N_PAD // NS
ACC_ROWS_PER_TILE = HALF // NS
E_PAD = 327680
EPT_DEG = E_PAD // NT
GRPE = 1024
RCAP = 11264
CHUNK = 64
NSLOT = 2
ROW_BLK = 1024

_mesh = plsc.VectorSubcoreMesh(core_axis_name="c", subcore_axis_name="s")


@functools.partial(
    pl.kernel,
    out_type=jax.ShapeDtypeStruct((NC * N_PAD,), jnp.float32),
    mesh=_mesh,
    scratch_types=[
        pltpu.VMEM((EPT_DEG // 128, 128), jnp.int32),
        pltpu.VMEM((128,), jnp.float32),
        pltpu.VMEM_SHARED((N_PAD,), jnp.float32),
    ],
)
def _deg_kernel(dst2d_hbm, z1d_hbm, deg_out_hbm, dst_v, ones_v, deg_sh):
    c = lax.axis_index("c")
    s = lax.axis_index("s")
    t = c * NS + s
    pltpu.sync_copy(z1d_hbm, deg_sh.at[pl.ds(s * ROWS_PER_TILE, ROWS_PER_TILE)])
    pltpu.sync_copy(
        dst2d_hbm.at[pl.ds(t * (EPT_DEG // 128), EPT_DEG // 128)], dst_v
    )
    for i in range(128 // LANES):
        ones_v[pl.ds(i * LANES, LANES)] = jnp.full((LANES,), 1.0, jnp.float32)
    plsc.subcore_barrier()

    def body(j, carry):
        pltpu.sync_copy(ones_v, deg_sh.at[dst_v.at[j]], add=True)
        return carry

    lax.fori_loop(0, EPT_DEG // 128, body, 0)
    plsc.subcore_barrier()
    pltpu.sync_copy(
        deg_sh.at[pl.ds(s * ROWS_PER_TILE, ROWS_PER_TILE)],
        deg_out_hbm.at[pl.ds(c * N_PAD + s * ROWS_PER_TILE, ROWS_PER_TILE)],
    )


@functools.partial(
    pl.kernel,
    out_type=[
        jax.ShapeDtypeStruct((2 * NT * RCAP,), jnp.int32),
        jax.ShapeDtypeStruct((2 * NT * RCAP,), jnp.int32),
        jax.ShapeDtypeStruct((2 * NT * 2 * RCAP,), jnp.int32),
        jax.ShapeDtypeStruct((2 * NT * 16,), jnp.int32),
    ],
    mesh=_mesh,
    compiler_params=pltpu.CompilerParams(needs_layout_passes=False),
    scratch_types=[
        pltpu.VMEM((EPT_DEG,), jnp.int32),
        pltpu.VMEM((EPT_DEG,), jnp.int32),
        pltpu.VMEM((RCAP,), jnp.int32),
        pltpu.VMEM((RCAP,), jnp.int32),
        pltpu.VMEM((RCAP,), jnp.int32),
        pltpu.VMEM((RCAP,), jnp.int32),
        pltpu.VMEM((2 * RCAP,), jnp.int32),
        pltpu.VMEM((2 * RCAP,), jnp.int32),
        pltpu.VMEM((16,), jnp.int32),
    ],
)
def _part_kernel(src_hbm, dst_hbm, bsrc_out, bdst_out, bdx2_out, cnt_out,
                 sin_v, din_v, b0s_v, b0d_v, b1s_v, b1d_v, b0x_v, b1x_v,
                 cnt_v):
    c = lax.axis_index("c")
    s = lax.axis_index("s")
    t = c * NS + s
    pltpu.sync_copy(src_hbm.at[pl.ds(t * EPT_DEG, EPT_DEG)], sin_v)
    pltpu.sync_copy(dst_hbm.at[pl.ds(t * EPT_DEG, EPT_DEG)], din_v)

    def body(i, carry):
        o0, o1 = carry
        sv = sin_v[pl.ds(i * LANES, LANES)]
        dv = din_v[pl.ds(i * LANES, LANES)]
        n0 = jnp.max(plsc.all_reduce_population_count(dv < HALF))
        dvs, svs = plsc.sort_key_val(dv, sv)
        lane = lax.iota(jnp.int32, LANES)
        m0s = lane < n0
        idx0 = o0 + lane
        idx1 = o1 + lane - n0
        plsc.store_scatter(b0s_v, [idx0], svs, mask=m0s)
        plsc.store_scatter(b0d_v, [idx0], dvs, mask=m0s)
        d2lo = dvs * 2
        plsc.store_scatter(b0x_v, [idx0 * 2], d2lo, mask=m0s)
        plsc.store_scatter(b0x_v, [idx0 * 2 + 1], d2lo + 1, mask=m0s)
        m1s = jnp.logical_not(m0s)
        dloc = dvs - HALF
        plsc.store_scatter(b1s_v, [idx1], svs, mask=m1s)
        plsc.store_scatter(b1d_v, [idx1], dloc, mask=m1s)
        d2lo1 = dloc * 2
        plsc.store_scatter(b1x_v, [idx1 * 2], d2lo1, mask=m1s)
        plsc.store_scatter(b1x_v, [idx1 * 2 + 1], d2lo1 + 1, mask=m1s)
        return (o0 + n0, o1 + (LANES - n0))

    o0, o1 = lax.fori_loop(0, EPT_DEG // LANES, body,
                           (jnp.int32(0), jnp.int32(0)))

    lane = lax.iota(jnp.int32, LANES)

    def padb(i, carry):
        po0, po1 = carry
        jsrc = N_NODES + lax.rem(lane + i * LANES, N_PAD - N_NODES)
        jdst = lane + i * LANES
        b0s_v[pl.ds(po0 + i * LANES, LANES)] = jsrc
        b0d_v[pl.ds(po0 + i * LANES, LANES)] = jdst
        b1s_v[pl.ds(po1 + i * LANES, LANES)] = jsrc
        b1d_v[pl.ds(po1 + i * LANES, LANES)] = jdst
        half = lane // 2
        par = lane - 2 * half
        jx_lo = 2 * (i * LANES) + 2 * half + par
        b0x_v[pl.ds(2 * po0 + 2 * i * LANES, LANES)] = jx_lo
        b0x_v[pl.ds(2 * po0 + 2 * i * LANES + LANES, LANES)] = jx_lo + LANES
        b1x_v[pl.ds(2 * po1 + 2 * i * LANES, LANES)] = jx_lo
        b1x_v[pl.ds(2 * po1 + 2 * i * LANES + LANES, LANES)] = jx_lo + LANES
        return carry

    lax.fori_loop(0, GRPE // LANES, padb, (o0, o1))
    p0 = ((o0 + GRPE - 1) // GRPE) * GRPE
    p1 = ((o1 + GRPE - 1) // GRPE) * GRPE

    cnt_v[...] = jnp.full((16,), 1, jnp.int32) * p0
    pltpu.sync_copy(cnt_v, cnt_out.at[pl.ds((0 * NT + t) * 16, 16)])
    cnt_v[...] = jnp.full((16,), 1, jnp.int32) * p1
    pltpu.sync_copy(cnt_v, cnt_out.at[pl.ds((1 * NT + t) * 16, 16)])
    pltpu.sync_copy(b0s_v, bsrc_out.at[pl.ds((0 * NT + t) * RCAP, RCAP)])
    pltpu.sync_copy(b0d_v, bdst_out.at[pl.ds((0 * NT + t) * RCAP, RCAP)])
    pltpu.sync_copy(b1s_v, bsrc_out.at[pl.ds((1 * NT + t) * RCAP, RCAP)])
    pltpu.sync_copy(b1d_v, bdst_out.at[pl.ds((1 * NT + t) * RCAP, RCAP)])
    pltpu.sync_copy(
        b0x_v, bdx2_out.at[pl.ds((0 * NT + t) * 2 * RCAP, 2 * RCAP)])
    pltpu.sync_copy(
        b1x_v, bdx2_out.at[pl.ds((1 * NT + t) * 2 * RCAP, 2 * RCAP)])


def _make_seg_sum(interleave):
    ipd = 2 if interleave else 1
    g_chunks = GRPE // CHUNK
    rounds = g_chunks // NSLOT

    @functools.partial(
        pl.kernel,
        out_type=jax.ShapeDtypeStruct((NC * ipd * HALF, 128), jnp.float32),
        mesh=_mesh,
        scratch_types=[
            pltpu.VMEM((GRPE,), jnp.int32),
            pltpu.VMEM((GRPE // CHUNK, ipd * CHUNK), jnp.int32),
            [pltpu.VMEM((ipd * CHUNK, 128), jnp.float32) for _ in range(NSLOT)],
            pltpu.VMEM((16,), jnp.int32),
            pltpu.VMEM_SHARED((ipd * HALF, 128), jnp.float32),
            [pltpu.SemaphoreType.DMA for _ in range(NSLOT)],
            [pltpu.SemaphoreType.DMA for _ in range(NSLOT)],
        ],
    )
    def seg(hs_hbm, bsrc_hbm, bdst2d_hbm, cnt_hbm, zw_hbm, acc_out_hbm,
            src_v, dst_v, rows, cnt_v, acc_sh, gsem, ssem):
        c = lax.axis_index("c")
        s = lax.axis_index("s")
        arpt = ipd * ACC_ROWS_PER_TILE
        pltpu.sync_copy(zw_hbm, acc_sh.at[pl.ds(s * arpt, arpt)])
        plsc.subcore_barrier()

        def gather(j, b):
            dst = rows[b].reshape(CHUNK, 2, 128) if interleave else rows[b]
            return pltpu.make_async_copy(
                hs_hbm.at[src_v.at[pl.ds(j * CHUNK, CHUNK)]], dst, gsem[b]
            )

        def scatter(j, b):
            return pltpu.make_async_copy(
                rows[b], acc_sh.at[dst_v.at[j]], ssem[b]
            )

        for r_i in range(2):
            r = 2 * s + r_i
            base_e = (c * NT + r) * RCAP
            base_row = (c * NT + r) * (RCAP // CHUNK)
            pltpu.sync_copy(cnt_hbm.at[pl.ds((c * NT + r) * 16, 16)], cnt_v)
            n_grp = cnt_v[...][0] // GRPE

            def group(g, carry):
                @pl.when(g < n_grp)
                def _():
                    pltpu.sync_copy(
                        bsrc_hbm.at[pl.ds(base_e + g * GRPE, GRPE)], src_v)
                    pltpu.sync_copy(
                        bdst2d_hbm.at[pl.ds(base_row + g * (GRPE // CHUNK),
                                            GRPE // CHUNK)], dst_v)
                    for b in range(NSLOT):
                        gather(b, b).start()

                    def body(k, carry2):
                        j0 = k * NSLOT
                        for b in range(NSLOT):
                            gather(j0 + b, b).wait()
                            scatter(j0 + b, b).start(add=True)

                        @pl.when(k < rounds - 1)
                        def _():
                            for b in range(NSLOT):
                                scatter(j0 + b, b).wait()
                                gather(j0 + NSLOT + b, b).start()
                        return carry2

                    lax.fori_loop(0, rounds, body, 0)
                    for b in range(NSLOT):
                        scatter(g_chunks - NSLOT + b, b).wait()
                return carry

            lax.fori_loop(0, RCAP // GRPE, group, 0)

        plsc.subcore_barrier()
        pltpu.sync_copy(
            acc_sh.at[pl.ds(s * arpt, arpt)],
            acc_out_hbm.at[pl.ds(c * ipd * HALF + s * arpt, arpt)],
        )

    return seg


_seg_sum_l1 = _make_seg_sum(True)
_seg_sum_l2 = _make_seg_sum(False)


def _mm1_body(x_ref, w1_ref, deg_ref, hs_ref, dis_ref):
    deg = deg_ref[0] + deg_ref[1] + 1.0
    dis = lax.rsqrt(deg)
    dis_ref[...] = dis
    h = jnp.dot(x_ref[...], w1_ref[...], preferred_element_type=jnp.float32)
    hs_ref[...] = h * dis[:, None]


def _mm2_body(acc_ref, hs_ref, dis_ref, b1_ref, w2_ref, hs2_ref):
    dis = dis_ref[...]
    h = jax.nn.relu(dis[:, None] * (acc_ref[...] + hs_ref[...])
                    + b1_ref[...][None, :])
    hs2 = jnp.dot(h, w2_ref[...], preferred_element_type=jnp.float32)
    hs2 = hs2 * dis[:, None]
    row = pl.program_id(0) * ROW_BLK + lax.broadcasted_iota(
        jnp.int32, (ROW_BLK, 1), 0)
    hs2_ref[...] = jnp.where(row < N_NODES, hs2, 0.0)


def _fin_body(acc_ref, hs_ref, dis_ref, b2_ref, out_ref):
    dis = dis_ref[...]
    out_ref[...] = (dis[:, None] * (acc_ref[...] + hs_ref[...])
                    + b2_ref[...][None, :])


def _row_grid():
    return N_PAD // ROW_BLK


def _tc_mm1(x_pad, W1, deg2):
    return pl.pallas_call(
        _mm1_body,
        grid=(_row_grid(),),
        in_specs=[
            pl.BlockSpec((ROW_BLK, IN_CH), lambda i: (i, 0)),
            pl.BlockSpec((IN_CH, 256), lambda i: (0, 0)),
            pl.BlockSpec((2, ROW_BLK), lambda i: (0, i)),
        ],
        out_specs=[
            pl.BlockSpec((ROW_BLK, 256), lambda i: (i, 0)),
            pl.BlockSpec((ROW_BLK,), lambda i: (i,)),
        ],
        out_shape=[
            jax.ShapeDtypeStruct((N_PAD, 256), jnp.float32),
            jax.ShapeDtypeStruct((N_PAD,), jnp.float32),
        ],
    )(x_pad, W1, deg2)


def _tc_mm2(acc1, hs1, dis, b1, W2):
    return pl.pallas_call(
        _mm2_body,
        grid=(_row_grid(),),
        in_specs=[
            pl.BlockSpec((ROW_BLK, 256), lambda i: (i, 0)),
            pl.BlockSpec((ROW_BLK, 256), lambda i: (i, 0)),
            pl.BlockSpec((ROW_BLK,), lambda i: (i,)),
            pl.BlockSpec((256,), lambda i: (0,)),
            pl.BlockSpec((256, 128), lambda i: (0, 0)),
        ],
        out_specs=pl.BlockSpec((ROW_BLK, 128), lambda i: (i, 0)),
        out_shape=jax.ShapeDtypeStruct((N_PAD, 128), jnp.float32),
    )(acc1, hs1, dis, b1, W2)


def _tc_fin(acc2, hs2, dis, b2):
    return pl.pallas_call(
        _fin_body,
        grid=(_row_grid(),),
        in_specs=[
            pl.BlockSpec((ROW_BLK, 128), lambda i: (i, 0)),
            pl.BlockSpec((ROW_BLK, 128), lambda i: (i, 0)),
            pl.BlockSpec((ROW_BLK,), lambda i: (i,)),
            pl.BlockSpec((OUT_CH,), lambda i: (0,)),
        ],
        out_specs=pl.BlockSpec((ROW_BLK, OUT_CH), lambda i: (i, 0)),
        out_shape=jax.ShapeDtypeStruct((N_PAD, OUT_CH), jnp.float32),
    )(acc2, hs2, dis, b2)


def kernel(x, edge_index, W1, b1, W2, b2):
    ei = edge_index.astype(jnp.int32)
    pad_e = E_PAD - N_EDGES
    pad_i = jnp.arange(pad_e, dtype=jnp.int32)
    src = jnp.concatenate([ei[0], N_NODES + pad_i % (N_PAD - N_NODES)])
    dst = jnp.concatenate([ei[1], pad_i % N_PAD])
    dst_deg = jnp.concatenate([ei[1], jnp.full((pad_e,), N_NODES, jnp.int32)])
    dst2d = dst_deg.reshape(E_PAD // 128, 128)

    x_pad = jnp.pad(x, ((0, N_PAD - N_NODES), (0, 0)))
    z1d = jnp.zeros((ROWS_PER_TILE,), jnp.float32)
    zl1 = jnp.zeros((2 * ACC_ROWS_PER_TILE, 128), jnp.float32)
    zl2 = jnp.zeros((ACC_ROWS_PER_TILE, 128), jnp.float32)

    bsrc, bdst, bdx2, cnts = _part_kernel(src, dst)
    bdst2d = bdst.reshape(2 * NT * RCAP // CHUNK, CHUNK)
    bdx2_2d = bdx2.reshape(2 * NT * 2 * RCAP // (2 * CHUNK), 2 * CHUNK)
    deg2 = _deg_kernel(dst2d, z1d).reshape(2, N_PAD)

    hs1, dis = _tc_mm1(x_pad, W1, deg2)
    hs1_3d = hs1.reshape(N_PAD, 2, 128)
    acc1 = _seg_sum_l1(hs1_3d, bsrc, bdx2_2d, cnts, zl1).reshape(N_PAD, 256)

    hs2 = _tc_mm2(acc1, hs1, dis, b1, W2)
    acc2 = _seg_sum_l2(hs2, bsrc, bdst2d, cnts, zl2)

    out = _tc_fin(acc2, hs2, dis, b2)
    return out[:N_NODES]


# speedup vs baseline: 5.8076x; 1.0918x over previous
import functools

import jax
import jax.numpy as jnp
from jax import lax
from jax.experimental import pallas as pl
from jax.experimental.pallas import tpu as pltpu
from jax.experimental.pallas import tpu_sc as plsc

N_NODES = 10000
N_EDGES = 320000
IN_CH = 128
OUT_CH = 128

NC, NS, LANES = 2, 16, 16
NT = NC * NS
N_PAD = 10240
HALF = N_PAD // 2
ROWS_PER_TILE = N_PAD // NS
ACC_ROWS_PER_TILE = HALF // NS
E_PAD = 327680
EPT_DEG = E_PAD // NT
GRPE = 1024
RCAP = 11264
CHUNK = 64
NSLOT = 2
ROW_BLK = 1024

_mesh = plsc.VectorSubcoreMesh(core_axis_name="c", subcore_axis_name="s")


@functools.partial(
    pl.kernel,
    out_type=jax.ShapeDtypeStruct((NC * N_PAD,), jnp.float32),
    mesh=_mesh,
    scratch_types=[
        pltpu.VMEM((EPT_DEG // 128, 128), jnp.int32),
        pltpu.VMEM((128,), jnp.float32),
        pltpu.VMEM_SHARED((N_PAD,), jnp.float32),
    ],
)
def _deg_kernel(dst2d_hbm, z1d_hbm, deg_out_hbm, dst_v, ones_v, deg_sh):
    c = lax.axis_index("c")
    s = lax.axis_index("s")
    t = c * NS + s
    pltpu.sync_copy(z1d_hbm, deg_sh.at[pl.ds(s * ROWS_PER_TILE, ROWS_PER_TILE)])
    pltpu.sync_copy(
        dst2d_hbm.at[pl.ds(t * (EPT_DEG // 128), EPT_DEG // 128)], dst_v
    )
    for i in range(128 // LANES):
        ones_v[pl.ds(i * LANES, LANES)] = jnp.full((LANES,), 1.0, jnp.float32)
    plsc.subcore_barrier()

    def body(j, carry):
        pltpu.sync_copy(ones_v, deg_sh.at[dst_v.at[j]], add=True)
        return carry

    lax.fori_loop(0, EPT_DEG // 128, body, 0)
    plsc.subcore_barrier()
    pltpu.sync_copy(
        deg_sh.at[pl.ds(s * ROWS_PER_TILE, ROWS_PER_TILE)],
        deg_out_hbm.at[pl.ds(c * N_PAD + s * ROWS_PER_TILE, ROWS_PER_TILE)],
    )


@functools.partial(
    pl.kernel,
    out_type=[
        jax.ShapeDtypeStruct((2 * NT * RCAP,), jnp.int32),
        jax.ShapeDtypeStruct((2 * NT * RCAP,), jnp.int32),
        jax.ShapeDtypeStruct((2 * NT * 2 * RCAP,), jnp.int32),
        jax.ShapeDtypeStruct((2 * NT * 16,), jnp.int32),
    ],
    mesh=_mesh,
    compiler_params=pltpu.CompilerParams(needs_layout_passes=False),
    scratch_types=[
        pltpu.VMEM((EPT_DEG,), jnp.int32),
        pltpu.VMEM((EPT_DEG,), jnp.int32),
        pltpu.VMEM((RCAP,), jnp.int32),
        pltpu.VMEM((RCAP,), jnp.int32),
        pltpu.VMEM((RCAP,), jnp.int32),
        pltpu.VMEM((RCAP,), jnp.int32),
        pltpu.VMEM((2 * RCAP,), jnp.int32),
        pltpu.VMEM((2 * RCAP,), jnp.int32),
        pltpu.VMEM((16,), jnp.int32),
    ],
)
def _part_kernel(src_hbm, dst_hbm, bsrc_out, bdst_out, bdx2_out, cnt_out,
                 sin_v, din_v, b0s_v, b0d_v, b1s_v, b1d_v, b0x_v, b1x_v,
                 cnt_v):
    c = lax.axis_index("c")
    s = lax.axis_index("s")
    t = c * NS + s
    pltpu.sync_copy(src_hbm.at[pl.ds(t * EPT_DEG, EPT_DEG)], sin_v)
    pltpu.sync_copy(dst_hbm.at[pl.ds(t * EPT_DEG, EPT_DEG)], din_v)

    def body(i, carry):
        o0, o1 = carry
        sv = sin_v[pl.ds(i * LANES, LANES)]
        dv = din_v[pl.ds(i * LANES, LANES)]
        n0 = jnp.max(plsc.all_reduce_population_count(dv < HALF))
        dvs, svs = plsc.sort_key_val(dv, sv)
        lane = lax.iota(jnp.int32, LANES)
        m0s = lane < n0
        idx0 = o0 + lane
        idx1 = o1 + lane - n0
        plsc.store_scatter(b0s_v, [idx0], svs, mask=m0s)
        plsc.store_scatter(b0d_v, [idx0], dvs, mask=m0s)
        d2lo = dvs * 2
        plsc.store_scatter(b0x_v, [idx0 * 2], d2lo, mask=m0s)
        plsc.store_scatter(b0x_v, [idx0 * 2 + 1], d2lo + 1, mask=m0s)
        m1s = jnp.logical_not(m0s)
        dloc = dvs - HALF
        plsc.store_scatter(b1s_v, [idx1], svs, mask=m1s)
        plsc.store_scatter(b1d_v, [idx1], dloc, mask=m1s)
        d2lo1 = dloc * 2
        plsc.store_scatter(b1x_v, [idx1 * 2], d2lo1, mask=m1s)
        plsc.store_scatter(b1x_v, [idx1 * 2 + 1], d2lo1 + 1, mask=m1s)
        return (o0 + n0, o1 + (LANES - n0))

    o0, o1 = lax.fori_loop(0, EPT_DEG // LANES, body,
                           (jnp.int32(0), jnp.int32(0)))

    lane = lax.iota(jnp.int32, LANES)

    def padb(i, carry):
        po0, po1 = carry
        jsrc = N_NODES + lax.rem(lane + i * LANES, N_PAD - N_NODES)
        jdst = lane + i * LANES
        b0s_v[pl.ds(po0 + i * LANES, LANES)] = jsrc
        b0d_v[pl.ds(po0 + i * LANES, LANES)] = jdst
        b1s_v[pl.ds(po1 + i * LANES, LANES)] = jsrc
        b1d_v[pl.ds(po1 + i * LANES, LANES)] = jdst
        half = lane // 2
        par = lane - 2 * half
        jx_lo = 2 * (i * LANES) + 2 * half + par
        b0x_v[pl.ds(2 * po0 + 2 * i * LANES, LANES)] = jx_lo
        b0x_v[pl.ds(2 * po0 + 2 * i * LANES + LANES, LANES)] = jx_lo + LANES
        b1x_v[pl.ds(2 * po1 + 2 * i * LANES, LANES)] = jx_lo
        b1x_v[pl.ds(2 * po1 + 2 * i * LANES + LANES, LANES)] = jx_lo + LANES
        return carry

    lax.fori_loop(0, GRPE // LANES, padb, (o0, o1))
    p0 = ((o0 + GRPE - 1) // GRPE) * GRPE
    p1 = ((o1 + GRPE - 1) // GRPE) * GRPE

    cnt_v[...] = jnp.full((16,), 1, jnp.int32) * p0
    pltpu.sync_copy(cnt_v, cnt_out.at[pl.ds((0 * NT + t) * 16, 16)])
    cnt_v[...] = jnp.full((16,), 1, jnp.int32) * p1
    pltpu.sync_copy(cnt_v, cnt_out.at[pl.ds((1 * NT + t) * 16, 16)])
    pltpu.sync_copy(b0s_v, bsrc_out.at[pl.ds((0 * NT + t) * RCAP, RCAP)])
    pltpu.sync_copy(b0d_v, bdst_out.at[pl.ds((0 * NT + t) * RCAP, RCAP)])
    pltpu.sync_copy(b1s_v, bsrc_out.at[pl.ds((1 * NT + t) * RCAP, RCAP)])
    pltpu.sync_copy(b1d_v, bdst_out.at[pl.ds((1 * NT + t) * RCAP, RCAP)])
    pltpu.sync_copy(
        b0x_v, bdx2_out.at[pl.ds((0 * NT + t) * 2 * RCAP, 2 * RCAP)])
    pltpu.sync_copy(
        b1x_v, bdx2_out.at[pl.ds((1 * NT + t) * 2 * RCAP, 2 * RCAP)])


def _make_seg_sum(interleave):
    ipd = 2 if interleave else 1
    nslot = NSLOT if interleave else 4
    g_chunks = GRPE // CHUNK
    rounds = g_chunks // nslot

    @functools.partial(
        pl.kernel,
        out_type=jax.ShapeDtypeStruct((NC * ipd * HALF, 128), jnp.float32),
        mesh=_mesh,
        scratch_types=[
            pltpu.VMEM((GRPE,), jnp.int32),
            pltpu.VMEM((GRPE // CHUNK, ipd * CHUNK), jnp.int32),
            [pltpu.VMEM((ipd * CHUNK, 128), jnp.float32) for _ in range(nslot)],
            pltpu.VMEM((16,), jnp.int32),
            pltpu.VMEM_SHARED((ipd * HALF, 128), jnp.float32),
            [pltpu.SemaphoreType.DMA for _ in range(nslot)],
            [pltpu.SemaphoreType.DMA for _ in range(nslot)],
        ],
    )
    def seg(hs_hbm, bsrc_hbm, bdst2d_hbm, cnt_hbm, zw_hbm, acc_out_hbm,
            src_v, dst_v, rows, cnt_v, acc_sh, gsem, ssem):
        c = lax.axis_index("c")
        s = lax.axis_index("s")
        arpt = ipd * ACC_ROWS_PER_TILE
        pltpu.sync_copy(zw_hbm, acc_sh.at[pl.ds(s * arpt, arpt)])
        plsc.subcore_barrier()

        def gather(j, b):
            dst = rows[b].reshape(CHUNK, 2, 128) if interleave else rows[b]
            return pltpu.make_async_copy(
                hs_hbm.at[src_v.at[pl.ds(j * CHUNK, CHUNK)]], dst, gsem[b]
            )

        def scatter(j, b):
            return pltpu.make_async_copy(
                rows[b], acc_sh.at[dst_v.at[j]], ssem[b]
            )

        for r_i in range(2):
            r = 2 * s + r_i
            base_e = (c * NT + r) * RCAP
            base_row = (c * NT + r) * (RCAP // CHUNK)
            pltpu.sync_copy(cnt_hbm.at[pl.ds((c * NT + r) * 16, 16)], cnt_v)
            n_grp = cnt_v[...][0] // GRPE

            def group(g, carry):
                @pl.when(g < n_grp)
                def _():
                    pltpu.sync_copy(
                        bsrc_hbm.at[pl.ds(base_e + g * GRPE, GRPE)], src_v)
                    pltpu.sync_copy(
                        bdst2d_hbm.at[pl.ds(base_row + g * (GRPE // CHUNK),
                                            GRPE // CHUNK)], dst_v)
                    for b in range(nslot):
                        gather(b, b).start()

                    def body(k, carry2):
                        j0 = k * nslot
                        for b in range(nslot):
                            gather(j0 + b, b).wait()
                            scatter(j0 + b, b).start(add=True)

                        @pl.when(k < rounds - 1)
                        def _():
                            for b in range(nslot):
                                scatter(j0 + b, b).wait()
                                gather(j0 + nslot + b, b).start()
                        return carry2

                    lax.fori_loop(0, rounds, body, 0)
                    for b in range(nslot):
                        scatter(g_chunks - nslot + b, b).wait()
                return carry

            lax.fori_loop(0, RCAP // GRPE, group, 0)

        plsc.subcore_barrier()
        pltpu.sync_copy(
            acc_sh.at[pl.ds(s * arpt, arpt)],
            acc_out_hbm.at[pl.ds(c * ipd * HALF + s * arpt, arpt)],
        )

    return seg


_seg_sum_l1 = _make_seg_sum(True)
_seg_sum_l2 = _make_seg_sum(False)


def _mm1_body(x_ref, w1_ref, deg_ref, hs_ref, dis_ref):
    deg = deg_ref[0] + deg_ref[1] + 1.0
    dis = lax.rsqrt(deg)
    dis_ref[...] = dis
    h = jnp.dot(x_ref[...], w1_ref[...], preferred_element_type=jnp.float32)
    hs_ref[...] = h * dis[:, None]


def _mm2_body(acc_ref, hs_ref, dis_ref, b1_ref, w2_ref, hs2_ref):
    dis = dis_ref[...]
    h = jax.nn.relu(dis[:, None] * (acc_ref[...] + hs_ref[...])
                    + b1_ref[...][None, :])
    hs2 = jnp.dot(h, w2_ref[...], preferred_element_type=jnp.float32)
    hs2 = hs2 * dis[:, None]
    row = pl.program_id(0) * ROW_BLK + lax.broadcasted_iota(
        jnp.int32, (ROW_BLK, 1), 0)
    hs2_ref[...] = jnp.where(row < N_NODES, hs2, 0.0)


def _fin_body(acc_ref, hs_ref, dis_ref, b2_ref, out_ref):
    dis = dis_ref[...]
    out_ref[...] = (dis[:, None] * (acc_ref[...] + hs_ref[...])
                    + b2_ref[...][None, :])


def _row_grid():
    return N_PAD // ROW_BLK


def _tc_mm1(x_pad, W1, deg2):
    return pl.pallas_call(
        _mm1_body,
        grid=(_row_grid(),),
        in_specs=[
            pl.BlockSpec((ROW_BLK, IN_CH), lambda i: (i, 0)),
            pl.BlockSpec((IN_CH, 256), lambda i: (0, 0)),
            pl.BlockSpec((2, ROW_BLK), lambda i: (0, i)),
        ],
        out_specs=[
            pl.BlockSpec((ROW_BLK, 256), lambda i: (i, 0)),
            pl.BlockSpec((ROW_BLK,), lambda i: (i,)),
        ],
        out_shape=[
            jax.ShapeDtypeStruct((N_PAD, 256), jnp.float32),
            jax.ShapeDtypeStruct((N_PAD,), jnp.float32),
        ],
    )(x_pad, W1, deg2)


def _tc_mm2(acc1, hs1, dis, b1, W2):
    return pl.pallas_call(
        _mm2_body,
        grid=(_row_grid(),),
        in_specs=[
            pl.BlockSpec((ROW_BLK, 256), lambda i: (i, 0)),
            pl.BlockSpec((ROW_BLK, 256), lambda i: (i, 0)),
            pl.BlockSpec((ROW_BLK,), lambda i: (i,)),
            pl.BlockSpec((256,), lambda i: (0,)),
            pl.BlockSpec((256, 128), lambda i: (0, 0)),
        ],
        out_specs=pl.BlockSpec((ROW_BLK, 128), lambda i: (i, 0)),
        out_shape=jax.ShapeDtypeStruct((N_PAD, 128), jnp.float32),
    )(acc1, hs1, dis, b1, W2)


def _tc_fin(acc2, hs2, dis, b2):
    return pl.pallas_call(
        _fin_body,
        grid=(_row_grid(),),
        in_specs=[
            pl.BlockSpec((ROW_BLK, 128), lambda i: (i, 0)),
            pl.BlockSpec((ROW_BLK, 128), lambda i: (i, 0)),
            pl.BlockSpec((ROW_BLK,), lambda i: (i,)),
            pl.BlockSpec((OUT_CH,), lambda i: (0,)),
        ],
        out_specs=pl.BlockSpec((ROW_BLK, OUT_CH), lambda i: (i, 0)),
        out_shape=jax.ShapeDtypeStruct((N_PAD, OUT_CH), jnp.float32),
    )(acc2, hs2, dis, b2)


def kernel(x, edge_index, W1, b1, W2, b2):
    ei = edge_index.astype(jnp.int32)
    pad_e = E_PAD - N_EDGES
    pad_i = jnp.arange(pad_e, dtype=jnp.int32)
    src = jnp.concatenate([ei[0], N_NODES + pad_i % (N_PAD - N_NODES)])
    dst = jnp.concatenate([ei[1], pad_i % N_PAD])
    dst_deg = jnp.concatenate([ei[1], jnp.full((pad_e,), N_NODES, jnp.int32)])
    dst2d = dst_deg.reshape(E_PAD // 128, 128)

    x_pad = jnp.pad(x, ((0, N_PAD - N_NODES), (0, 0)))
    z1d = jnp.zeros((ROWS_PER_TILE,), jnp.float32)
    zl1 = jnp.zeros((2 * ACC_ROWS_PER_TILE, 128), jnp.float32)
    zl2 = jnp.zeros((ACC_ROWS_PER_TILE, 128), jnp.float32)

    bsrc, bdst, bdx2, cnts = _part_kernel(src, dst)
    bdst2d = bdst.reshape(2 * NT * RCAP // CHUNK, CHUNK)
    bdx2_2d = bdx2.reshape(2 * NT * 2 * RCAP // (2 * CHUNK), 2 * CHUNK)
    deg2 = _deg_kernel(dst2d, z1d).reshape(2, N_PAD)

    hs1, dis = _tc_mm1(x_pad, W1, deg2)
    hs1_3d = hs1.reshape(N_PAD, 2, 128)
    acc1 = _seg_sum_l1(hs1_3d, bsrc, bdx2_2d, cnts, zl1).reshape(N_PAD, 256)

    hs2 = _tc_mm2(acc1, hs1, dis, b1, W2)
    acc2 = _seg_sum_l2(hs2, bsrc, bdst2d, cnts, zl2)

    out = _tc_fin(acc2, hs2, dis, b2)
    return out[:N_NODES]
